# baseline (speedup 1.0000x reference)
import functools
from math import pi as PI

import jax
import jax.numpy as jnp
from jax import lax
from jax.experimental import pallas as pl
from jax.experimental.pallas import tpu as pltpu
from jax.experimental.pallas import tpu_sc as plsc

N = 10000
E = 320000
HC = 128
NG = 50
NGP = 64
NI = 6
CUTOFF = 10.0
BOX = 5.0
HALF = BOX / 2.0
LOG2 = 0.6931471805599453

T = 4096
DMAX = 4.3302
HTAB = DMAX / (T - 1)

NC, NS, L = 2, 16, 16
NW = NC * NS
HH = HC // NC
EPT = E // NW
KE = 256
NKSUB = KE // 128
NCH = E // KE
NROW = 624
NROWL = N - NROW * (NS - 1)


def _ssp(x):
    return jnp.maximum(x, 0.0) + jnp.log(1.0 + jnp.exp(-jnp.abs(x))) - LOG2


@functools.lru_cache(maxsize=None)
def _mesh():
    return plsc.VectorSubcoreMesh(
        core_axis_name="c", subcore_axis_name="s",
        num_cores=NC, num_subcores=NS)


def _phase0_body(px_hbm, py_hbm, pz_hbm, row_hbm, col_hbm,
                 t_hbm,
                 px, py, pz, rbuf, cbuf, tbuf):
    c = lax.axis_index("c")
    s = lax.axis_index("s")
    wid = s * NC + c
    base = pl.multiple_of(wid * EPT, EPT)
    pltpu.sync_copy(px_hbm, px)
    pltpu.sync_copy(py_hbm, py)
    pltpu.sync_copy(pz_hbm, pz)
    pltpu.sync_copy(row_hbm.at[pl.ds(base, EPT)], rbuf)
    pltpu.sync_copy(col_hbm.at[pl.ds(base, EPT)], cbuf)

    def mic(a, b):
        d = a - b
        d = jnp.where(d > HALF, d - BOX, jnp.where(d < -HALF, d + BOX, d))
        return d

    def body(i, carry):
        sl = pl.ds(pl.multiple_of(i * L, L), L)
        ir = rbuf[sl]
        ic = cbuf[sl]
        dx = mic(plsc.load_gather(px, [ir]), plsc.load_gather(px, [ic]))
        dy = mic(plsc.load_gather(py, [ir]), plsc.load_gather(py, [ic]))
        dz = mic(plsc.load_gather(pz, [ir]), plsc.load_gather(pz, [ic]))
        u = dx * dx + dy * dy + dz * dz + 1e-12
        yi = jnp.int32(0x1FBD1DF5) + (plsc.bitcast(u, jnp.int32) >> 1)
        y = plsc.bitcast(yi, jnp.float32)
        y = 0.5 * (y + u / y)
        y = 0.5 * (y + u / y)
        y = 0.5 * (y + u / y)
        sc = y * (1.0 / HTAB) + 0.5
        tbuf[sl] = jnp.minimum(sc.astype(jnp.int32), T - 1)
        return carry

    lax.fori_loop(0, EPT // L, body, 0)
    pltpu.sync_copy(tbuf, t_hbm.at[pl.ds(base, EPT)])


@functools.lru_cache(maxsize=None)
def _phase0_fn():
    return pl.kernel(
        _phase0_body,
        out_type=jax.ShapeDtypeStruct((E,), jnp.int32),
        mesh=_mesh(),
        compiler_params=pltpu.CompilerParams(needs_layout_passes=False, use_tc_tiling_on_sc=False),
        scratch_types=[
            pltpu.VMEM((N,), jnp.float32),
            pltpu.VMEM((N,), jnp.float32),
            pltpu.VMEM((N,), jnp.float32),
            pltpu.VMEM((EPT,), jnp.int32),
            pltpu.VMEM((EPT,), jnp.int32),
            pltpu.VMEM((EPT,), jnp.int32),
        ],
    )


def _block_body(x1_hbm, tab_hbm, idx_hbm, z_hbm,
                out_hbm,
                xb0, xb1, wb0, wb1, pb0, pb1, aggsh, sem0, sem1):
    c = lax.axis_index("c")
    s = lax.axis_index("s")

    @pl.when(s < NS - 1)
    def _():
        pltpu.sync_copy(z_hbm.at[pl.ds(0, NROW)],
                        aggsh.at[pl.ds(s * NROW, NROW)])

    @pl.when(s == NS - 1)
    def _():
        pltpu.sync_copy(z_hbm, aggsh.at[pl.ds((NS - 1) * NROW, NROWL)])

    plsc.subcore_barrier()

    xbufs, wbufs, pbufs = (xb0, xb1), (wb0, wb1), (pb0, pb1)
    sems = (sem0, sem1)

    def issue(ci, p):
        pltpu.sync_copy(idx_hbm.at[ci], pbufs[p])
        for j in range(NKSUB):
            pltpu.async_copy(x1_hbm.at[c].at[pbufs[p].at[0, j]],
                             xbufs[p].at[pl.ds(j * 128, 128)], sems[p])
            pltpu.async_copy(tab_hbm.at[c].at[pbufs[p].at[2, j]],
                             wbufs[p].at[pl.ds(j * 128, 128)], sems[p])

    def finish(p):
        for j in range(NKSUB):
            pltpu.make_async_copy(x1_hbm.at[c].at[pbufs[p].at[0, j]],
                                  xbufs[p].at[pl.ds(j * 128, 128)],
                                  sems[p]).wait()
            pltpu.make_async_copy(tab_hbm.at[c].at[pbufs[p].at[2, j]],
                                  wbufs[p].at[pl.ds(j * 128, 128)],
                                  sems[p]).wait()

        def mul(e, mcarry):
            for j in range(HH // L):
                sl = pl.ds(j * L, L)
                xbufs[p][e, sl] = xbufs[p][e, sl] * wbufs[p][e, sl]
            return mcarry

        lax.fori_loop(0, KE, mul, 0, unroll=8)
        for j in range(NKSUB):
            pltpu.sync_copy(xbufs[p].at[pl.ds(j * 128, 128)],
                            aggsh.at[pbufs[p].at[1, j]], add=True)

    nkm = (NCH // NS) // 2 * 2

    issue(s, 0)

    def body(q, carry):
        k0 = 2 * q
        issue((k0 + 1) * NS + s, 1)
        finish(0)

        @pl.when(q < nkm // 2 - 1)
        def _():
            issue((k0 + 2) * NS + s, 0)

        finish(1)
        return carry

    lax.fori_loop(0, nkm // 2, body, 0)

    @pl.when(s < NCH - nkm * NS)
    def _():
        issue(nkm * NS + s, 0)
        finish(0)

    plsc.subcore_barrier()

    @pl.when(s < NS - 1)
    def _():
        pltpu.sync_copy(aggsh.at[pl.ds(s * NROW, NROW)],
                        out_hbm.at[c, pl.ds(s * NROW, NROW)])

    @pl.when(s == NS - 1)
    def _():
        pltpu.sync_copy(aggsh.at[pl.ds((NS - 1) * NROW, NROWL)],
                        out_hbm.at[c, pl.ds((NS - 1) * NROW, NROWL)])


@functools.lru_cache(maxsize=None)
def _block_fn():
    return pl.kernel(
        _block_body,
        out_type=jax.ShapeDtypeStruct((NC, N, HH), jnp.float32),
        mesh=_mesh(),
        compiler_params=pltpu.CompilerParams(needs_layout_passes=False, use_tc_tiling_on_sc=False),
        scratch_types=[
            pltpu.VMEM((KE, HH), jnp.float32),
            pltpu.VMEM((KE, HH), jnp.float32),
            pltpu.VMEM((KE, HH), jnp.float32),
            pltpu.VMEM((KE, HH), jnp.float32),
            pltpu.VMEM((3, NKSUB, 128), jnp.int32),
            pltpu.VMEM((3, NKSUB, 128), jnp.int32),
            pltpu.VMEM_SHARED((N, HH), jnp.float32),
            pltpu.SemaphoreType.DMA,
            pltpu.SemaphoreType.DMA,
        ],
    )


def _tables_body(w1_ref, b1_ref, w2_ref, b2_ref, out_ref):
    dk = lax.broadcasted_iota(jnp.int32, (T, NGP), 0).astype(jnp.float32) * HTAB
    kk = lax.broadcasted_iota(jnp.int32, (T, NGP), 1).astype(jnp.float32)
    delta = CUTOFF / (NG - 1)
    coeff = -0.5 / (delta * delta)
    g = jnp.where(kk < float(NG), jnp.exp(coeff * (dk - kk * delta) ** 2), 0.0)
    dcol = dk[:, :1]
    env = 0.5 * (jnp.cos(dcol * (PI / CUTOFF)) + 1.0)
    for b in range(NI):
        mid = _ssp(jnp.dot(g, w1_ref[b], preferred_element_type=jnp.float32)
                   + b1_ref[b])
        tab = (jnp.dot(mid, w2_ref[b], preferred_element_type=jnp.float32)
               + b2_ref[b]) * env
        out_ref[b, 0] = tab[:, :HH]
        out_ref[b, 1] = tab[:, HH:]


def _tables(w1s, b1s, w2s, b2s):
    return pl.pallas_call(
        _tables_body,
        out_shape=jax.ShapeDtypeStruct((NI, NC, T, HH), jnp.float32),
    )(w1s, b1s, w2s, b2s)


def _pre_body(z_ref, emb_ref, cw1_ref, h_ref, x1_ref):
    zz = z_ref[...]
    h0 = jnp.where(zz == 0, emb_ref[0:1, :], emb_ref[1:2, :])
    h_ref[...] = h0
    x1 = jnp.dot(h0, cw1_ref[...], preferred_element_type=jnp.float32)
    x1_ref[0] = x1[:, :HH]
    x1_ref[1] = x1[:, HH:]


def _pre(zc, emb, cw1):
    return pl.pallas_call(
        _pre_body,
        out_shape=(jax.ShapeDtypeStruct((N, HC), jnp.float32),
                   jax.ShapeDtypeStruct((NC, N, HH), jnp.float32)),
    )(zc, emb, cw1)


def _blk_body(h_ref, agg_ref, cw2_ref, cb2_ref, lw_ref, lb_ref, cw1n_ref,
              hn_ref, x1n_ref):
    agg = jnp.concatenate([agg_ref[0], agg_ref[1]], axis=1)
    x2 = _ssp(jnp.dot(agg, cw2_ref[...], preferred_element_type=jnp.float32)
              + cb2_ref[...])
    x2 = jnp.dot(x2, lw_ref[...], preferred_element_type=jnp.float32) + lb_ref[...]
    hn = h_ref[...] + x2
    hn_ref[...] = hn
    x1 = jnp.dot(hn, cw1n_ref[...], preferred_element_type=jnp.float32)
    x1n_ref[0] = x1[:, :HH]
    x1n_ref[1] = x1[:, HH:]


def _tc_block(h, aggs, cw2, cb2, lw, lb, cw1n):
    return pl.pallas_call(
        _blk_body,
        out_shape=(jax.ShapeDtypeStruct((N, HC), jnp.float32),
                   jax.ShapeDtypeStruct((NC, N, HH), jnp.float32)),
    )(h, aggs, cw2, cb2, lw, lb, cw1n)


def _fin_body(h_ref, agg_ref, cw2_ref, cb2_ref, lw_ref, lb_ref,
              l1w_ref, l1b_ref, l2w_ref, l2b_ref, out_ref):
    agg = jnp.concatenate([agg_ref[0], agg_ref[1]], axis=1)
    x2 = _ssp(jnp.dot(agg, cw2_ref[...], preferred_element_type=jnp.float32)
              + cb2_ref[...])
    x2 = jnp.dot(x2, lw_ref[...], preferred_element_type=jnp.float32) + lb_ref[...]
    hn = h_ref[...] + x2
    r = _ssp(jnp.dot(hn, l1w_ref[...], preferred_element_type=jnp.float32)
             + l1b_ref[...])
    r2 = jnp.dot(r, l2w_ref[...], preferred_element_type=jnp.float32) + l2b_ref[...]
    out_ref[...] = jnp.sum(r2).reshape(1, 1)


def _tc_final(h, aggs, cw2, cb2, lw, lb, l1w, l1b, l2w, l2b):
    return pl.pallas_call(
        _fin_body,
        out_shape=jax.ShapeDtypeStruct((1, 1), jnp.float32),
    )(h, aggs, cw2, cb2, lw, lb, l1w, l1b, l2w, l2b)


def kernel(z, pos, edge_index, params):
    row = edge_index[0].astype(jnp.int32)
    col = edge_index[1].astype(jnp.int32)
    posx = pos[:, 0]
    posy = pos[:, 1]
    posz = pos[:, 2]
    blocks = params['blocks']

    w1s = jnp.stack([jnp.pad(b['mlp_w1'], ((0, NGP - NG), (0, 0)))
                     for b in blocks])
    b1s = jnp.stack([b['mlp_b1'].reshape(1, HC) for b in blocks])
    w2s = jnp.stack([b['mlp_w2'] for b in blocks])
    b2s = jnp.stack([b['mlp_b2'].reshape(1, HC) for b in blocks])
    tabs = _tables(w1s, b1s, w2s, b2s)

    tarr = _phase0_fn()(posx, posy, posz, row, col)

    idx_pack = jnp.stack([row.reshape(NCH, NKSUB, 128),
                          col.reshape(NCH, NKSUB, 128),
                          tarr.reshape(NCH, NKSUB, 128)], axis=1)
    zrows = jnp.zeros((NROWL, HH), jnp.float32)

    zc = z.astype(jnp.int32).reshape(N, 1)
    h, x1 = _pre(zc, params['emb'], blocks[0]['cw1'])

    out = None
    for b in range(NI):
        bb = blocks[b]
        aggs = _block_fn()(x1, tabs[b], idx_pack, zrows)
        cb2 = bb['cb2'].reshape(1, HC)
        lb = bb['lb'].reshape(1, HC)
        if b < NI - 1:
            h, x1 = _tc_block(h, aggs, bb['cw2'], cb2, bb['lw'], lb,
                              blocks[b + 1]['cw1'])
        else:
            out = _tc_final(h, aggs, bb['cw2'], cb2, bb['lw'], lb,
                            params['l1w'], params['l1b'].reshape(1, HC // 2),
                            params['l2w'], params['l2b'].reshape(1, 1))
    return out


# speedup vs baseline: 2.0559x; 2.0559x over previous
import functools
from math import pi as PI

import jax
import jax.numpy as jnp
from jax import lax
from jax.experimental import pallas as pl
from jax.experimental.pallas import tpu as pltpu
from jax.experimental.pallas import tpu_sc as plsc

N = 10000
E = 320000
HC = 128
NG = 50
NGP = 64
NI = 6
CUTOFF = 10.0
BOX = 5.0
HALF = BOX / 2.0
LOG2 = 0.6931471805599453

T = 2048
DMAX = 4.3302
HTAB = DMAX / (T - 1)

NC, NS, L = 2, 16, 16
NW = NC * NS
HH = HC // NC
EPT = E // NW
KE = 256
NKSUB = KE // 128
NCH = E // KE
NROW = 624
NROWL = N - NROW * (NS - 1)


def _ssp(x):
    return jnp.maximum(x, 0.0) + jnp.log(1.0 + jnp.exp(-jnp.abs(x))) - LOG2


@functools.lru_cache(maxsize=None)
def _mesh():
    return plsc.VectorSubcoreMesh(
        core_axis_name="c", subcore_axis_name="s",
        num_cores=NC, num_subcores=NS)


def _phase0_body(px_hbm, py_hbm, pz_hbm, row_hbm, col_hbm,
                 t_hbm,
                 px, py, pz, rbuf, cbuf, tbuf):
    c = lax.axis_index("c")
    s = lax.axis_index("s")
    wid = s * NC + c
    base = pl.multiple_of(wid * EPT, EPT)
    pltpu.sync_copy(px_hbm, px)
    pltpu.sync_copy(py_hbm, py)
    pltpu.sync_copy(pz_hbm, pz)
    pltpu.sync_copy(row_hbm.at[pl.ds(base, EPT)], rbuf)
    pltpu.sync_copy(col_hbm.at[pl.ds(base, EPT)], cbuf)

    def mic(a, b):
        d = a - b
        d = jnp.where(d > HALF, d - BOX, jnp.where(d < -HALF, d + BOX, d))
        return d

    def body(i, carry):
        sl = pl.ds(pl.multiple_of(i * L, L), L)
        ir = rbuf[sl]
        ic = cbuf[sl]
        dx = mic(plsc.load_gather(px, [ir]), plsc.load_gather(px, [ic]))
        dy = mic(plsc.load_gather(py, [ir]), plsc.load_gather(py, [ic]))
        dz = mic(plsc.load_gather(pz, [ir]), plsc.load_gather(pz, [ic]))
        u = dx * dx + dy * dy + dz * dz + 1e-12
        yi = jnp.int32(0x1FBD1DF5) + (plsc.bitcast(u, jnp.int32) >> 1)
        y = plsc.bitcast(yi, jnp.float32)
        y = 0.5 * (y + u / y)
        y = 0.5 * (y + u / y)
        y = 0.5 * (y + u / y)
        sc = y * (1.0 / HTAB) + 0.5
        tbuf[sl] = jnp.minimum(sc.astype(jnp.int32), T - 1)
        return carry

    lax.fori_loop(0, EPT // L, body, 0)
    pltpu.sync_copy(tbuf, t_hbm.at[pl.ds(base, EPT)])


@functools.lru_cache(maxsize=None)
def _phase0_fn():
    return pl.kernel(
        _phase0_body,
        out_type=jax.ShapeDtypeStruct((E,), jnp.int32),
        mesh=_mesh(),
        compiler_params=pltpu.CompilerParams(needs_layout_passes=False, use_tc_tiling_on_sc=False),
        scratch_types=[
            pltpu.VMEM((N,), jnp.float32),
            pltpu.VMEM((N,), jnp.float32),
            pltpu.VMEM((N,), jnp.float32),
            pltpu.VMEM((EPT,), jnp.int32),
            pltpu.VMEM((EPT,), jnp.int32),
            pltpu.VMEM((EPT,), jnp.int32),
        ],
    )


def _block_body(x1_hbm, tab_hbm, idx_hbm, z_hbm,
                out_hbm,
                xb0, xb1, wb0, wb1, pb0, pb1, aggsh, sem0, sem1):
    c = lax.axis_index("c")
    s = lax.axis_index("s")

    @pl.when(s < NS - 1)
    def _():
        pltpu.sync_copy(z_hbm.at[pl.ds(0, NROW)],
                        aggsh.at[pl.ds(s * NROW, NROW)])

    @pl.when(s == NS - 1)
    def _():
        pltpu.sync_copy(z_hbm, aggsh.at[pl.ds((NS - 1) * NROW, NROWL)])

    plsc.subcore_barrier()

    xbufs, wbufs, pbufs = (xb0, xb1), (wb0, wb1), (pb0, pb1)
    sems = (sem0, sem1)

    def issue(ci, p):
        pltpu.sync_copy(idx_hbm.at[ci], pbufs[p])
        for j in range(NKSUB):
            pltpu.async_copy(x1_hbm.at[c].at[pbufs[p].at[0, j]],
                             xbufs[p].at[pl.ds(j * 128, 128)], sems[p])
            pltpu.async_copy(tab_hbm.at[c].at[pbufs[p].at[2, j]],
                             wbufs[p].at[pl.ds(j * 128, 128)], sems[p])

    def finish(p):
        for j in range(NKSUB):
            pltpu.make_async_copy(x1_hbm.at[c].at[pbufs[p].at[0, j]],
                                  xbufs[p].at[pl.ds(j * 128, 128)],
                                  sems[p]).wait()
            pltpu.make_async_copy(tab_hbm.at[c].at[pbufs[p].at[2, j]],
                                  wbufs[p].at[pl.ds(j * 128, 128)],
                                  sems[p]).wait()

        def mul(e, mcarry):
            for j in range(HH // L):
                sl = pl.ds(j * L, L)
                xbufs[p][e, sl] = xbufs[p][e, sl] * wbufs[p][e, sl]
            return mcarry

        lax.fori_loop(0, KE, mul, 0)
        for j in range(NKSUB):
            pltpu.sync_copy(xbufs[p].at[pl.ds(j * 128, 128)],
                            aggsh.at[pbufs[p].at[1, j]], add=True)

    nkm = (NCH // NS) // 2 * 2

    issue(s, 0)

    def body(q, carry):
        k0 = 2 * q
        issue((k0 + 1) * NS + s, 1)
        finish(0)

        @pl.when(q < nkm // 2 - 1)
        def _():
            issue((k0 + 2) * NS + s, 0)

        finish(1)
        return carry

    lax.fori_loop(0, nkm // 2, body, 0)

    @pl.when(s < NCH - nkm * NS)
    def _():
        issue(nkm * NS + s, 0)
        finish(0)

    plsc.subcore_barrier()

    @pl.when(s < NS - 1)
    def _():
        pltpu.sync_copy(aggsh.at[pl.ds(s * NROW, NROW)],
                        out_hbm.at[c, pl.ds(s * NROW, NROW)])

    @pl.when(s == NS - 1)
    def _():
        pltpu.sync_copy(aggsh.at[pl.ds((NS - 1) * NROW, NROWL)],
                        out_hbm.at[c, pl.ds((NS - 1) * NROW, NROWL)])


@functools.lru_cache(maxsize=None)
def _block_fn():
    return pl.kernel(
        _block_body,
        out_type=jax.ShapeDtypeStruct((NC, N, HH), jnp.float32),
        mesh=_mesh(),
        compiler_params=pltpu.CompilerParams(needs_layout_passes=False, use_tc_tiling_on_sc=False),
        scratch_types=[
            pltpu.VMEM((KE, HH), jnp.float32),
            pltpu.VMEM((KE, HH), jnp.float32),
            pltpu.VMEM((KE, HH), jnp.float32),
            pltpu.VMEM((KE, HH), jnp.float32),
            pltpu.VMEM((3, NKSUB, 128), jnp.int32),
            pltpu.VMEM((3, NKSUB, 128), jnp.int32),
            pltpu.VMEM_SHARED((N, HH), jnp.float32),
            pltpu.SemaphoreType.DMA,
            pltpu.SemaphoreType.DMA,
        ],
    )


def _tables_body(w1_ref, b1_ref, w2_ref, b2_ref, out_ref):
    dk = lax.broadcasted_iota(jnp.int32, (T, NGP), 0).astype(jnp.float32) * HTAB
    kk = lax.broadcasted_iota(jnp.int32, (T, NGP), 1).astype(jnp.float32)
    delta = CUTOFF / (NG - 1)
    coeff = -0.5 / (delta * delta)
    g = jnp.where(kk < float(NG), jnp.exp(coeff * (dk - kk * delta) ** 2), 0.0)
    dcol = dk[:, :1]
    env = 0.5 * (jnp.cos(dcol * (PI / CUTOFF)) + 1.0)
    for b in range(NI):
        mid = _ssp(jnp.dot(g, w1_ref[b], preferred_element_type=jnp.float32)
                   + b1_ref[b])
        tab = (jnp.dot(mid, w2_ref[b], preferred_element_type=jnp.float32)
               + b2_ref[b]) * env
        out_ref[b, 0] = tab[:, :HH]
        out_ref[b, 1] = tab[:, HH:]


def _tables(w1s, b1s, w2s, b2s):
    return pl.pallas_call(
        _tables_body,
        out_shape=jax.ShapeDtypeStruct((NI, NC, T, HH), jnp.float32),
    )(w1s, b1s, w2s, b2s)


def _pre_body(z_ref, emb_ref, cw1_ref, h_ref, x1_ref):
    zz = z_ref[...]
    h0 = jnp.where(zz == 0, emb_ref[0:1, :], emb_ref[1:2, :])
    h_ref[...] = h0
    x1 = jnp.dot(h0, cw1_ref[...], preferred_element_type=jnp.float32)
    x1_ref[0] = x1[:, :HH]
    x1_ref[1] = x1[:, HH:]


def _pre(zc, emb, cw1):
    return pl.pallas_call(
        _pre_body,
        out_shape=(jax.ShapeDtypeStruct((N, HC), jnp.float32),
                   jax.ShapeDtypeStruct((NC, N, HH), jnp.float32)),
    )(zc, emb, cw1)


def _blk_body(h_ref, agg_ref, cw2_ref, cb2_ref, lw_ref, lb_ref, cw1n_ref,
              hn_ref, x1n_ref):
    agg = jnp.concatenate([agg_ref[0], agg_ref[1]], axis=1)
    x2 = _ssp(jnp.dot(agg, cw2_ref[...], preferred_element_type=jnp.float32)
              + cb2_ref[...])
    x2 = jnp.dot(x2, lw_ref[...], preferred_element_type=jnp.float32) + lb_ref[...]
    hn = h_ref[...] + x2
    hn_ref[...] = hn
    x1 = jnp.dot(hn, cw1n_ref[...], preferred_element_type=jnp.float32)
    x1n_ref[0] = x1[:, :HH]
    x1n_ref[1] = x1[:, HH:]


def _tc_block(h, aggs, cw2, cb2, lw, lb, cw1n):
    return pl.pallas_call(
        _blk_body,
        out_shape=(jax.ShapeDtypeStruct((N, HC), jnp.float32),
                   jax.ShapeDtypeStruct((NC, N, HH), jnp.float32)),
    )(h, aggs, cw2, cb2, lw, lb, cw1n)


def _fin_body(h_ref, agg_ref, cw2_ref, cb2_ref, lw_ref, lb_ref,
              l1w_ref, l1b_ref, l2w_ref, l2b_ref, out_ref):
    agg = jnp.concatenate([agg_ref[0], agg_ref[1]], axis=1)
    x2 = _ssp(jnp.dot(agg, cw2_ref[...], preferred_element_type=jnp.float32)
              + cb2_ref[...])
    x2 = jnp.dot(x2, lw_ref[...], preferred_element_type=jnp.float32) + lb_ref[...]
    hn = h_ref[...] + x2
    r = _ssp(jnp.dot(hn, l1w_ref[...], preferred_element_type=jnp.float32)
             + l1b_ref[...])
    r2 = jnp.dot(r, l2w_ref[...], preferred_element_type=jnp.float32) + l2b_ref[...]
    out_ref[...] = jnp.sum(r2).reshape(1, 1)


def _tc_final(h, aggs, cw2, cb2, lw, lb, l1w, l1b, l2w, l2b):
    return pl.pallas_call(
        _fin_body,
        out_shape=jax.ShapeDtypeStruct((1, 1), jnp.float32),
    )(h, aggs, cw2, cb2, lw, lb, l1w, l1b, l2w, l2b)


def kernel(z, pos, edge_index, params):
    row = edge_index[0].astype(jnp.int32)
    col = edge_index[1].astype(jnp.int32)
    posx = pos[:, 0]
    posy = pos[:, 1]
    posz = pos[:, 2]
    blocks = params['blocks']

    w1s = jnp.stack([jnp.pad(b['mlp_w1'], ((0, NGP - NG), (0, 0)))
                     for b in blocks])
    b1s = jnp.stack([b['mlp_b1'].reshape(1, HC) for b in blocks])
    w2s = jnp.stack([b['mlp_w2'] for b in blocks])
    b2s = jnp.stack([b['mlp_b2'].reshape(1, HC) for b in blocks])
    tabs = _tables(w1s, b1s, w2s, b2s)

    tarr = _phase0_fn()(posx, posy, posz, row, col)

    idx_pack = jnp.stack([row.reshape(NCH, NKSUB, 128),
                          col.reshape(NCH, NKSUB, 128),
                          tarr.reshape(NCH, NKSUB, 128)], axis=1)
    zrows = jnp.zeros((NROWL, HH), jnp.float32)

    zc = z.astype(jnp.int32).reshape(N, 1)
    h, x1 = _pre(zc, params['emb'], blocks[0]['cw1'])

    out = None
    for b in range(NI):
        bb = blocks[b]
        aggs = _block_fn()(x1, tabs[b], idx_pack, zrows)
        cb2 = bb['cb2'].reshape(1, HC)
        lb = bb['lb'].reshape(1, HC)
        if b < NI - 1:
            h, x1 = _tc_block(h, aggs, bb['cw2'], cb2, bb['lw'], lb,
                              blocks[b + 1]['cw1'])
        else:
            out = _tc_final(h, aggs, bb['cw2'], cb2, bb['lw'], lb,
                            params['l1w'], params['l1b'].reshape(1, HC // 2),
                            params['l2w'], params['l2b'].reshape(1, 1))
    return out
